# Initial kernel scaffold; baseline (speedup 1.0000x reference)
import functools

import jax
import jax.numpy as jnp
from jax import lax
from jax.experimental import pallas as pl
from jax.experimental.pallas import tpu as pltpu
from jax.experimental.pallas import tpu_sc as plsc

N = 10000
E = 320000
D_IN = 256
H = 16

NC = 2
NS = 16
NW = NC * NS
L = 16

EPAD = 327680
EPT = EPAD // NW
ROWS_PT = EPT // 128
CHUNKS = 10
ROWS_PC = ROWS_PT // CHUNKS
CE = ROWS_PC * 128
NACC = 10240
SLICE = NACC // NS

_mesh = plsc.VectorSubcoreMesh(core_axis_name="c", subcore_axis_name="s")


def _wid():
    return lax.axis_index("s") * NC + lax.axis_index("c")


@functools.partial(
    pl.kernel,
    out_type=jax.ShapeDtypeStruct((NW, NACC), jnp.float32),
    mesh=_mesh,
    scratch_types=[
        pltpu.VMEM((EPT // L, L), jnp.int32),
        pltpu.VMEM((NACC,), jnp.float32),
    ],
)
def _k_deg(dst16_hbm, deg_out_hbm, dstbuf, hist):
    wid = _wid()
    pltpu.sync_copy(dst16_hbm.at[pl.ds(wid * (EPT // L), EPT // L)], dstbuf)
    zero = jnp.zeros((L,), jnp.float32)
    one = jnp.ones((L,), jnp.float32)

    def zbody(i, _):
        hist[pl.ds(i * L, L)] = zero
        return 0
    lax.fori_loop(0, NACC // L, zbody, 0)

    def abody(g, _):
        idx = dstbuf[g]
        plsc.addupdate_scatter(hist, [idx], one)
        return 0
    lax.fori_loop(0, EPT // L, abody, 0)

    pltpu.sync_copy(hist, deg_out_hbm.at[wid])


@functools.partial(
    pl.kernel,
    out_type=jax.ShapeDtypeStruct((NC, NACC, H), jnp.float32),
    mesh=_mesh,
    scratch_types=[
        pltpu.VMEM((ROWS_PC, 128), jnp.int32),
        pltpu.VMEM((ROWS_PC, 128), jnp.int32),
        pltpu.VMEM((CE, H), jnp.float32),
        pltpu.VMEM((SLICE, H), jnp.float32),
        pltpu.VMEM_SHARED((NACC, H), jnp.float32),
        pltpu.SemaphoreType.DMA,
    ],
)
def _k_scat(xs_hbm, src2d_hbm, dst2d_hbm, acc_out_hbm,
            src_v, dst_v, rows_v, zbuf, acc_sh, sem):
    cid = lax.axis_index("c")
    sid = lax.axis_index("s")
    wid = sid * NC + cid
    zero = jnp.zeros((L,), jnp.float32)

    def zbody(i, _):
        zbuf[i] = zero
        return 0
    lax.fori_loop(0, SLICE, zbody, 0)
    pltpu.sync_copy(zbuf, acc_sh.at[pl.ds(sid * SLICE, SLICE)])
    plsc.subcore_barrier()

    def chunk(g, _):
        rbase = wid * ROWS_PT + g * ROWS_PC
        pltpu.sync_copy(src2d_hbm.at[pl.ds(rbase, ROWS_PC)], src_v)
        pltpu.sync_copy(dst2d_hbm.at[pl.ds(rbase, ROWS_PC)], dst_v)
        descs = [
            pltpu.async_copy(xs_hbm.at[src_v.at[j]],
                             rows_v.at[pl.ds(j * 128, 128)], sem)
            for j in range(ROWS_PC)
        ]
        for d in descs:
            d.wait()
        for j in range(ROWS_PC):
            pltpu.sync_copy(rows_v.at[pl.ds(j * 128, 128)],
                            acc_sh.at[dst_v.at[j]], add=True)
        return 0
    lax.fori_loop(0, CHUNKS, chunk, 0)

    plsc.subcore_barrier()
    pltpu.sync_copy(acc_sh.at[pl.ds(sid * SLICE, SLICE)],
                    acc_out_hbm.at[cid].at[pl.ds(sid * SLICE, SLICE)])


def _k_dinv_body(degT_ref, dinv_ref):
    deg = jnp.sum(degT_ref[...], axis=1, keepdims=True) + 1.0
    dinv_ref[...] = jnp.broadcast_to(lax.rsqrt(deg), dinv_ref.shape)


def _dinv_call(degT):
    blk = 1024
    return pl.pallas_call(
        _k_dinv_body,
        grid=(NACC // blk,),
        in_specs=[pl.BlockSpec((blk, NW), lambda i: (i, 0))],
        out_specs=pl.BlockSpec((blk, H), lambda i: (i, 0)),
        out_shape=jax.ShapeDtypeStruct((NACC, H), jnp.float32),
    )(degT)


def _k_dense1_body(x_ref, w_ref, dinv_ref, xs_ref):
    xw = jnp.dot(x_ref[...], w_ref[...], preferred_element_type=jnp.float32)
    xs_ref[...] = xw * dinv_ref[...]


def _dense1_call(x, W1, dinv):
    blk = 1000
    return pl.pallas_call(
        _k_dense1_body,
        grid=(N // blk,),
        in_specs=[
            pl.BlockSpec((blk, D_IN), lambda i: (i, 0)),
            pl.BlockSpec((D_IN, H), lambda i: (0, 0)),
            pl.BlockSpec((blk, H), lambda i: (i, 0)),
        ],
        out_specs=pl.BlockSpec((blk, H), lambda i: (i, 0)),
        out_shape=jax.ShapeDtypeStruct((N, H), jnp.float32),
    )(x, W1, dinv)


def _k_dense2_body(acc_ref, xs1_ref, dinv_ref, b1_ref, w2_ref, xs2_ref):
    a = acc_ref[...]
    dinv = dinv_ref[...]
    h = dinv * (a[0] + a[1] + xs1_ref[...]) + b1_ref[...]
    h = jnp.maximum(h, 0.0)
    xs2_ref[...] = jnp.dot(h, w2_ref[...],
                           preferred_element_type=jnp.float32) * dinv


def _dense2_call(acc, xs1, dinv, b1, W2):
    blk = 1000
    return pl.pallas_call(
        _k_dense2_body,
        grid=(N // blk,),
        in_specs=[
            pl.BlockSpec((NC, blk, H), lambda i: (0, i, 0)),
            pl.BlockSpec((blk, H), lambda i: (i, 0)),
            pl.BlockSpec((blk, H), lambda i: (i, 0)),
            pl.BlockSpec((1, H), lambda i: (0, 0)),
            pl.BlockSpec((H, H), lambda i: (0, 0)),
        ],
        out_specs=pl.BlockSpec((blk, H), lambda i: (i, 0)),
        out_shape=jax.ShapeDtypeStruct((N, H), jnp.float32),
    )(acc, xs1, dinv, b1, W2)


def _k_dense3_body(acc_ref, xs2_ref, dinv_ref, b2_ref, out_ref):
    a = acc_ref[...]
    out_ref[...] = dinv_ref[...] * (a[0] + a[1] + xs2_ref[...]) + b2_ref[...]


def _dense3_call(acc, xs2, dinv, b2):
    blk = 1000
    return pl.pallas_call(
        _k_dense3_body,
        grid=(N // blk,),
        in_specs=[
            pl.BlockSpec((NC, blk, H), lambda i: (0, i, 0)),
            pl.BlockSpec((blk, H), lambda i: (i, 0)),
            pl.BlockSpec((blk, H), lambda i: (i, 0)),
            pl.BlockSpec((1, H), lambda i: (0, 0)),
        ],
        out_specs=pl.BlockSpec((blk, H), lambda i: (i, 0)),
        out_shape=jax.ShapeDtypeStruct((N, H), jnp.float32),
    )(acc, xs2, dinv, b2)


def kernel(x, coo, W1, b1, W2, b2):
    src = coo[:, 0]
    dst = coo[:, 1]
    npad = EPAD - E
    src_p = jnp.concatenate([src, jnp.zeros((npad,), jnp.int32)])
    dst_p = jnp.concatenate([dst, jnp.full((npad,), N, jnp.int32)])
    src2d = src_p.reshape(EPAD // 128, 128)
    dst2d = dst_p.reshape(EPAD // 128, 128)
    dst16 = dst_p.reshape(EPAD // L, L)

    deg_parts = _k_deg(dst16)
    dinv = _dinv_call(deg_parts.T)
    dinv_n = dinv[:N]

    xs1 = _dense1_call(x, W1, dinv_n)
    acc1 = _k_scat(xs1, src2d, dst2d)
    xs2 = _dense2_call(acc1, xs1, dinv_n, b1.reshape(1, H), W2)
    acc2 = _k_scat(xs2, src2d, dst2d)
    out = _dense3_call(acc2, xs2, dinv_n, b2.reshape(1, H))
    return out


# speedup vs baseline: 30.0911x; 30.0911x over previous
import functools

import jax
import jax.numpy as jnp
from jax import lax
from jax.experimental import pallas as pl
from jax.experimental.pallas import tpu as pltpu
from jax.experimental.pallas import tpu_sc as plsc

N = 10000
E = 320000
D_IN = 256
H = 16

NC = 2
NS = 16
NW = NC * NS
L = 16

EPAD = 327680
EPT = EPAD // NW
ROWS_PT = EPT // 128
CHUNKS = 10
ROWS_PC = ROWS_PT // CHUNKS
CE = ROWS_PC * 128
NACC = 10240
SLICE = NACC // NS

_mesh = plsc.VectorSubcoreMesh(core_axis_name="c", subcore_axis_name="s")
_sc_params = pltpu.CompilerParams(use_tc_tiling_on_sc=False)


def _wid():
    return lax.axis_index("s") * NC + lax.axis_index("c")


@functools.partial(
    pl.kernel,
    out_type=jax.ShapeDtypeStruct((NC, NACC), jnp.float32),
    mesh=_mesh,
    scratch_types=[
        pltpu.VMEM((ROWS_PC, 128), jnp.int32),
        pltpu.VMEM((128,), jnp.float32),
        pltpu.VMEM((SLICE,), jnp.float32),
        pltpu.VMEM_SHARED((NACC,), jnp.float32),
    ],
    compiler_params=_sc_params,
)
def _k_deg(dst2d_hbm, deg_out_hbm, dst_v, ones_v, zbuf, deg_sh):
    cid = lax.axis_index("c")
    sid = lax.axis_index("s")
    wid = sid * NC + cid
    zero = jnp.zeros((L,), jnp.float32)
    one = jnp.ones((L,), jnp.float32)

    def zbody(i, _):
        zbuf[pl.ds(i * L, L)] = zero
        return 0
    lax.fori_loop(0, SLICE // L, zbody, 0)
    for j in range(128 // L):
        ones_v[pl.ds(j * L, L)] = one
    pltpu.sync_copy(zbuf, deg_sh.at[pl.ds(sid * SLICE, SLICE)])
    plsc.subcore_barrier()

    def chunk(g, _):
        rbase = wid * ROWS_PT + g * ROWS_PC
        pltpu.sync_copy(dst2d_hbm.at[pl.ds(rbase, ROWS_PC)], dst_v)
        for j in range(ROWS_PC):
            pltpu.sync_copy(ones_v, deg_sh.at[dst_v.at[j]], add=True)
        return 0
    lax.fori_loop(0, CHUNKS, chunk, 0)

    plsc.subcore_barrier()
    pltpu.sync_copy(deg_sh.at[pl.ds(sid * SLICE, SLICE)],
                    deg_out_hbm.at[cid].at[pl.ds(sid * SLICE, SLICE)])


@functools.partial(
    pl.kernel,
    out_type=jax.ShapeDtypeStruct((NC, NACC, H), jnp.float32),
    mesh=_mesh,
    scratch_types=[
        pltpu.VMEM((ROWS_PC, 128), jnp.int32),
        pltpu.VMEM((ROWS_PC, 128), jnp.int32),
        pltpu.VMEM((CE, H), jnp.float32),
        pltpu.VMEM((SLICE, H), jnp.float32),
        pltpu.VMEM_SHARED((NACC, H), jnp.float32),
        pltpu.SemaphoreType.DMA,
    ],
    compiler_params=_sc_params,
)
def _k_scat(xs_hbm, src2d_hbm, dst2d_hbm, acc_out_hbm,
            src_v, dst_v, rows_v, zbuf, acc_sh, sem):
    cid = lax.axis_index("c")
    sid = lax.axis_index("s")
    wid = sid * NC + cid
    zero = jnp.zeros((L,), jnp.float32)

    def zbody(i, _):
        zbuf[i] = zero
        return 0
    lax.fori_loop(0, SLICE, zbody, 0)
    pltpu.sync_copy(zbuf, acc_sh.at[pl.ds(sid * SLICE, SLICE)])
    plsc.subcore_barrier()

    def chunk(g, _):
        rbase = wid * ROWS_PT + g * ROWS_PC
        pltpu.sync_copy(src2d_hbm.at[pl.ds(rbase, ROWS_PC)], src_v)
        pltpu.sync_copy(dst2d_hbm.at[pl.ds(rbase, ROWS_PC)], dst_v)
        descs = [
            pltpu.async_copy(xs_hbm.at[src_v.at[j]],
                             rows_v.at[pl.ds(j * 128, 128)], sem)
            for j in range(ROWS_PC)
        ]
        for d in descs:
            d.wait()
        for j in range(ROWS_PC):
            pltpu.sync_copy(rows_v.at[pl.ds(j * 128, 128)],
                            acc_sh.at[dst_v.at[j]], add=True)
        return 0
    lax.fori_loop(0, CHUNKS, chunk, 0)

    plsc.subcore_barrier()
    pltpu.sync_copy(acc_sh.at[pl.ds(sid * SLICE, SLICE)],
                    acc_out_hbm.at[cid].at[pl.ds(sid * SLICE, SLICE)])


def _k_dinv_body(degT_ref, dinv_ref):
    deg = jnp.sum(degT_ref[...], axis=1, keepdims=True) + 1.0
    dinv_ref[...] = jnp.broadcast_to(lax.rsqrt(deg), dinv_ref.shape)


def _dinv_call(degT):
    blk = 1024
    return pl.pallas_call(
        _k_dinv_body,
        grid=(NACC // blk,),
        in_specs=[pl.BlockSpec((blk, NC), lambda i: (i, 0))],
        out_specs=pl.BlockSpec((blk, H), lambda i: (i, 0)),
        out_shape=jax.ShapeDtypeStruct((NACC, H), jnp.float32),
    )(degT)


def _k_dense1_body(x_ref, w_ref, dinv_ref, xs_ref):
    xw = jnp.dot(x_ref[...], w_ref[...], preferred_element_type=jnp.float32)
    xs_ref[...] = xw * dinv_ref[...]


def _dense1_call(x, W1, dinv):
    blk = 1000
    return pl.pallas_call(
        _k_dense1_body,
        grid=(N // blk,),
        in_specs=[
            pl.BlockSpec((blk, D_IN), lambda i: (i, 0)),
            pl.BlockSpec((D_IN, H), lambda i: (0, 0)),
            pl.BlockSpec((blk, H), lambda i: (i, 0)),
        ],
        out_specs=pl.BlockSpec((blk, H), lambda i: (i, 0)),
        out_shape=jax.ShapeDtypeStruct((N, H), jnp.float32),
    )(x, W1, dinv)


def _k_dense2_body(acc_ref, xs1_ref, dinv_ref, b1_ref, w2_ref, xs2_ref):
    a = acc_ref[...]
    dinv = dinv_ref[...]
    h = dinv * (a[0] + a[1] + xs1_ref[...]) + b1_ref[...]
    h = jnp.maximum(h, 0.0)
    xs2_ref[...] = jnp.dot(h, w2_ref[...],
                           preferred_element_type=jnp.float32) * dinv


def _dense2_call(acc, xs1, dinv, b1, W2):
    blk = 1000
    return pl.pallas_call(
        _k_dense2_body,
        grid=(N // blk,),
        in_specs=[
            pl.BlockSpec((NC, blk, H), lambda i: (0, i, 0)),
            pl.BlockSpec((blk, H), lambda i: (i, 0)),
            pl.BlockSpec((blk, H), lambda i: (i, 0)),
            pl.BlockSpec((1, H), lambda i: (0, 0)),
            pl.BlockSpec((H, H), lambda i: (0, 0)),
        ],
        out_specs=pl.BlockSpec((blk, H), lambda i: (i, 0)),
        out_shape=jax.ShapeDtypeStruct((N, H), jnp.float32),
    )(acc, xs1, dinv, b1, W2)


def _k_dense3_body(acc_ref, xs2_ref, dinv_ref, b2_ref, out_ref):
    a = acc_ref[...]
    out_ref[...] = dinv_ref[...] * (a[0] + a[1] + xs2_ref[...]) + b2_ref[...]


def _dense3_call(acc, xs2, dinv, b2):
    blk = 1000
    return pl.pallas_call(
        _k_dense3_body,
        grid=(N // blk,),
        in_specs=[
            pl.BlockSpec((NC, blk, H), lambda i: (0, i, 0)),
            pl.BlockSpec((blk, H), lambda i: (i, 0)),
            pl.BlockSpec((blk, H), lambda i: (i, 0)),
            pl.BlockSpec((1, H), lambda i: (0, 0)),
        ],
        out_specs=pl.BlockSpec((blk, H), lambda i: (i, 0)),
        out_shape=jax.ShapeDtypeStruct((N, H), jnp.float32),
    )(acc, xs2, dinv, b2)


def kernel(x, coo, W1, b1, W2, b2):
    src = coo[:, 0]
    dst = coo[:, 1]
    npad = EPAD - E
    src_p = jnp.concatenate([src, jnp.zeros((npad,), jnp.int32)])
    dst_p = jnp.concatenate([dst, jnp.full((npad,), N, jnp.int32)])
    src2d = src_p.reshape(EPAD // 128, 128)
    dst2d = dst_p.reshape(EPAD // 128, 128)

    deg_parts = _k_deg(dst2d)
    dinv = _dinv_call(deg_parts.T)
    dinv_n = dinv[:N]

    xs1 = _dense1_call(x, W1, dinv_n)
    acc1 = _k_scat(xs1, src2d, dst2d)
    xs2 = _dense2_call(acc1, xs1, dinv_n, b1.reshape(1, H), W2)
    acc2 = _k_scat(xs2, src2d, dst2d)
    out = _dense3_call(acc2, xs2, dinv_n, b2.reshape(1, H))
    return out


# speedup vs baseline: 34.2486x; 1.1382x over previous
import functools

import jax
import jax.numpy as jnp
from jax import lax
from jax.experimental import pallas as pl
from jax.experimental.pallas import tpu as pltpu
from jax.experimental.pallas import tpu_sc as plsc

N = 10000
E = 320000
D_IN = 256
H = 16

NC = 2
NS = 16
NW = NC * NS
L = 16

EPAD = 327680
EPT = EPAD // NW
ROWS_PT = EPT // 128
CHUNKS = 10
ROWS_PC = ROWS_PT // CHUNKS
CE = ROWS_PC * 128
NACC = 10240
SLICE = NACC // NS

_mesh = plsc.VectorSubcoreMesh(core_axis_name="c", subcore_axis_name="s")
_sc_params = pltpu.CompilerParams(use_tc_tiling_on_sc=False)


def _wid():
    return lax.axis_index("s") * NC + lax.axis_index("c")


@functools.partial(
    pl.kernel,
    out_type=jax.ShapeDtypeStruct((NC, NACC), jnp.float32),
    mesh=_mesh,
    scratch_types=[
        pltpu.VMEM((ROWS_PC, 128), jnp.int32),
        pltpu.VMEM((128,), jnp.float32),
        pltpu.VMEM((SLICE,), jnp.float32),
        pltpu.VMEM_SHARED((NACC,), jnp.float32),
    ],
    compiler_params=_sc_params,
)
def _k_deg(dst2d_hbm, deg_out_hbm, dst_v, ones_v, zbuf, deg_sh):
    cid = lax.axis_index("c")
    sid = lax.axis_index("s")
    wid = sid * NC + cid
    zero = jnp.zeros((L,), jnp.float32)
    one = jnp.ones((L,), jnp.float32)

    def zbody(i, _):
        zbuf[pl.ds(i * L, L)] = zero
        return 0
    lax.fori_loop(0, SLICE // L, zbody, 0)
    for j in range(128 // L):
        ones_v[pl.ds(j * L, L)] = one
    pltpu.sync_copy(zbuf, deg_sh.at[pl.ds(sid * SLICE, SLICE)])
    plsc.subcore_barrier()

    def chunk(g, _):
        rbase = wid * ROWS_PT + g * ROWS_PC
        pltpu.sync_copy(dst2d_hbm.at[pl.ds(rbase, ROWS_PC)], dst_v)
        for j in range(ROWS_PC):
            pltpu.sync_copy(ones_v, deg_sh.at[dst_v.at[j]], add=True)
        return 0
    lax.fori_loop(0, CHUNKS, chunk, 0)

    plsc.subcore_barrier()
    pltpu.sync_copy(deg_sh.at[pl.ds(sid * SLICE, SLICE)],
                    deg_out_hbm.at[cid].at[pl.ds(sid * SLICE, SLICE)])


@functools.partial(
    pl.kernel,
    out_type=jax.ShapeDtypeStruct((NC, NACC, H), jnp.float32),
    mesh=_mesh,
    scratch_types=[
        pltpu.VMEM((2, ROWS_PC, 128), jnp.int32),
        pltpu.VMEM((2, ROWS_PC, 128), jnp.int32),
        pltpu.VMEM((2, CE, H), jnp.float32),
        pltpu.VMEM((SLICE, H), jnp.float32),
        pltpu.VMEM_SHARED((NACC, H), jnp.float32),
        pltpu.SemaphoreType.DMA,
        pltpu.SemaphoreType.DMA,
    ],
    compiler_params=_sc_params,
)
def _k_scat(xs_hbm, src2d_hbm, dst2d_hbm, acc_out_hbm,
            src_v, dst_v, rows_v, zbuf, acc_sh, sem_a, sem_b):
    cid = lax.axis_index("c")
    sid = lax.axis_index("s")
    wid = sid * NC + cid
    zero = jnp.zeros((L,), jnp.float32)
    sems = (sem_a, sem_b)

    def zbody(i, _):
        zbuf[i] = zero
        return 0
    lax.fori_loop(0, SLICE, zbody, 0)
    pltpu.sync_copy(zbuf, acc_sh.at[pl.ds(sid * SLICE, SLICE)])
    plsc.subcore_barrier()

    def stage_and_fire(g, b):
        rbase = wid * ROWS_PT + g * ROWS_PC
        pltpu.sync_copy(src2d_hbm.at[pl.ds(rbase, ROWS_PC)], src_v.at[b])
        pltpu.sync_copy(dst2d_hbm.at[pl.ds(rbase, ROWS_PC)], dst_v.at[b])
        return [
            pltpu.async_copy(xs_hbm.at[src_v.at[b].at[j]],
                             rows_v.at[b].at[pl.ds(j * 128, 128)], sems[b])
            for j in range(ROWS_PC)
        ]

    descs = {0: stage_and_fire(0, 0)}
    for g in range(CHUNKS):
        b = g & 1
        if g + 1 < CHUNKS:
            descs[g + 1] = stage_and_fire(g + 1, 1 - b)
        for d in descs.pop(g):
            d.wait()
        for j in range(ROWS_PC):
            pltpu.sync_copy(rows_v.at[b].at[pl.ds(j * 128, 128)],
                            acc_sh.at[dst_v.at[b].at[j]], add=True)

    plsc.subcore_barrier()
    pltpu.sync_copy(acc_sh.at[pl.ds(sid * SLICE, SLICE)],
                    acc_out_hbm.at[cid].at[pl.ds(sid * SLICE, SLICE)])


def _k_dinv_body(degT_ref, dinv_ref):
    deg = jnp.sum(degT_ref[...], axis=1, keepdims=True) + 1.0
    dinv_ref[...] = jnp.broadcast_to(lax.rsqrt(deg), dinv_ref.shape)


def _dinv_call(degT):
    blk = 1024
    return pl.pallas_call(
        _k_dinv_body,
        grid=(NACC // blk,),
        in_specs=[pl.BlockSpec((blk, NC), lambda i: (i, 0))],
        out_specs=pl.BlockSpec((blk, H), lambda i: (i, 0)),
        out_shape=jax.ShapeDtypeStruct((NACC, H), jnp.float32),
    )(degT)


def _k_dense1_body(x_ref, w_ref, dinv_ref, xs_ref):
    xw = jnp.dot(x_ref[...], w_ref[...], preferred_element_type=jnp.float32)
    xs_ref[...] = xw * dinv_ref[...]


def _dense1_call(x, W1, dinv):
    blk = 1000
    return pl.pallas_call(
        _k_dense1_body,
        grid=(N // blk,),
        in_specs=[
            pl.BlockSpec((blk, D_IN), lambda i: (i, 0)),
            pl.BlockSpec((D_IN, H), lambda i: (0, 0)),
            pl.BlockSpec((blk, H), lambda i: (i, 0)),
        ],
        out_specs=pl.BlockSpec((blk, H), lambda i: (i, 0)),
        out_shape=jax.ShapeDtypeStruct((N, H), jnp.float32),
    )(x, W1, dinv)


def _k_dense2_body(acc_ref, xs1_ref, dinv_ref, b1_ref, w2_ref, xs2_ref):
    a = acc_ref[...]
    dinv = dinv_ref[...]
    h = dinv * (a[0] + a[1] + xs1_ref[...]) + b1_ref[...]
    h = jnp.maximum(h, 0.0)
    xs2_ref[...] = jnp.dot(h, w2_ref[...],
                           preferred_element_type=jnp.float32) * dinv


def _dense2_call(acc, xs1, dinv, b1, W2):
    blk = 1000
    return pl.pallas_call(
        _k_dense2_body,
        grid=(N // blk,),
        in_specs=[
            pl.BlockSpec((NC, blk, H), lambda i: (0, i, 0)),
            pl.BlockSpec((blk, H), lambda i: (i, 0)),
            pl.BlockSpec((blk, H), lambda i: (i, 0)),
            pl.BlockSpec((1, H), lambda i: (0, 0)),
            pl.BlockSpec((H, H), lambda i: (0, 0)),
        ],
        out_specs=pl.BlockSpec((blk, H), lambda i: (i, 0)),
        out_shape=jax.ShapeDtypeStruct((N, H), jnp.float32),
    )(acc, xs1, dinv, b1, W2)


def _k_dense3_body(acc_ref, xs2_ref, dinv_ref, b2_ref, out_ref):
    a = acc_ref[...]
    out_ref[...] = dinv_ref[...] * (a[0] + a[1] + xs2_ref[...]) + b2_ref[...]


def _dense3_call(acc, xs2, dinv, b2):
    blk = 1000
    return pl.pallas_call(
        _k_dense3_body,
        grid=(N // blk,),
        in_specs=[
            pl.BlockSpec((NC, blk, H), lambda i: (0, i, 0)),
            pl.BlockSpec((blk, H), lambda i: (i, 0)),
            pl.BlockSpec((blk, H), lambda i: (i, 0)),
            pl.BlockSpec((1, H), lambda i: (0, 0)),
        ],
        out_specs=pl.BlockSpec((blk, H), lambda i: (i, 0)),
        out_shape=jax.ShapeDtypeStruct((N, H), jnp.float32),
    )(acc, xs2, dinv, b2)


def kernel(x, coo, W1, b1, W2, b2):
    src = coo[:, 0]
    dst = coo[:, 1]
    npad = EPAD - E
    src_p = jnp.concatenate([src, jnp.zeros((npad,), jnp.int32)])
    dst_p = jnp.concatenate([dst, jnp.full((npad,), N, jnp.int32)])
    src2d = src_p.reshape(EPAD // 128, 128)
    dst2d = dst_p.reshape(EPAD // 128, 128)

    deg_parts = _k_deg(dst2d)
    dinv = _dinv_call(deg_parts.T)
    dinv_n = dinv[:N]

    xs1 = _dense1_call(x, W1, dinv_n)
    acc1 = _k_scat(xs1, src2d, dst2d)
    xs2 = _dense2_call(acc1, xs1, dinv_n, b1.reshape(1, H), W2)
    acc2 = _k_scat(xs2, src2d, dst2d)
    out = _dense3_call(acc2, xs2, dinv_n, b2.reshape(1, H))
    return out


# speedup vs baseline: 35.1266x; 1.0256x over previous
import functools

import jax
import jax.numpy as jnp
from jax import lax
from jax.experimental import pallas as pl
from jax.experimental.pallas import tpu as pltpu
from jax.experimental.pallas import tpu_sc as plsc

N = 10000
E = 320000
D_IN = 256
H = 16

NC = 2
NS = 16
NW = NC * NS
L = 16

EPAD = 327680
EPT = EPAD // NW
ROWS_PT = EPT // 128
CHUNKS = 10
ROWS_PC = ROWS_PT // CHUNKS
CE = ROWS_PC * 128
NACC = 10240
SLICE = NACC // NS

_mesh = plsc.VectorSubcoreMesh(core_axis_name="c", subcore_axis_name="s")
_sc_params = pltpu.CompilerParams(use_tc_tiling_on_sc=False)


def _wid():
    return lax.axis_index("s") * NC + lax.axis_index("c")


@functools.partial(
    pl.kernel,
    out_type=jax.ShapeDtypeStruct((NC, NACC), jnp.float32),
    mesh=_mesh,
    scratch_types=[
        pltpu.VMEM((ROWS_PT, 128), jnp.int32),
        pltpu.VMEM((128,), jnp.float32),
        pltpu.VMEM((SLICE,), jnp.float32),
        pltpu.VMEM_SHARED((NACC,), jnp.float32),
        pltpu.SemaphoreType.DMA,
    ],
    compiler_params=_sc_params,
)
def _k_deg(dst2d_hbm, deg_out_hbm, dst_v, ones_v, zbuf, deg_sh, sem):
    cid = lax.axis_index("c")
    sid = lax.axis_index("s")
    wid = sid * NC + cid
    zero = jnp.zeros((L,), jnp.float32)
    one = jnp.ones((L,), jnp.float32)

    pltpu.sync_copy(dst2d_hbm.at[pl.ds(wid * ROWS_PT, ROWS_PT)], dst_v)

    def zbody(i, _):
        zbuf[pl.ds(i * L, L)] = zero
        return 0
    lax.fori_loop(0, SLICE // L, zbody, 0)
    for j in range(128 // L):
        ones_v[pl.ds(j * L, L)] = one
    pltpu.sync_copy(zbuf, deg_sh.at[pl.ds(sid * SLICE, SLICE)])
    plsc.subcore_barrier()

    descs = [
        pltpu.async_copy(ones_v, deg_sh.at[dst_v.at[j]], sem, add=True)
        for j in range(ROWS_PT)
    ]
    for d in descs:
        d.wait()

    plsc.subcore_barrier()
    pltpu.sync_copy(deg_sh.at[pl.ds(sid * SLICE, SLICE)],
                    deg_out_hbm.at[cid].at[pl.ds(sid * SLICE, SLICE)])


@functools.partial(
    pl.kernel,
    out_type=jax.ShapeDtypeStruct((NC, NACC, H), jnp.float32),
    mesh=_mesh,
    scratch_types=[
        pltpu.VMEM((2, ROWS_PC, 128), jnp.int32),
        pltpu.VMEM((2, ROWS_PC, 128), jnp.int32),
        pltpu.VMEM((2, CE, H), jnp.float32),
        pltpu.VMEM((SLICE, H), jnp.float32),
        pltpu.VMEM_SHARED((NACC, H), jnp.float32),
        pltpu.SemaphoreType.DMA,
        pltpu.SemaphoreType.DMA,
        pltpu.SemaphoreType.DMA,
        pltpu.SemaphoreType.DMA,
    ],
    compiler_params=_sc_params,
)
def _k_scat(xs_hbm, src2d_hbm, dst2d_hbm, acc_out_hbm,
            src_v, dst_v, rows_v, zbuf, acc_sh,
            sem_ga, sem_gb, sem_sa, sem_sb):
    cid = lax.axis_index("c")
    sid = lax.axis_index("s")
    wid = sid * NC + cid
    zero = jnp.zeros((L,), jnp.float32)
    sems = (sem_ga, sem_gb)
    sems_s = (sem_sa, sem_sb)

    def zbody(i, _):
        zbuf[i] = zero
        return 0
    lax.fori_loop(0, SLICE, zbody, 0)
    pltpu.sync_copy(zbuf, acc_sh.at[pl.ds(sid * SLICE, SLICE)])
    plsc.subcore_barrier()

    def stage_and_fire(g, b):
        rbase = wid * ROWS_PT + g * ROWS_PC
        pltpu.sync_copy(src2d_hbm.at[pl.ds(rbase, ROWS_PC)], src_v.at[b])
        pltpu.sync_copy(dst2d_hbm.at[pl.ds(rbase, ROWS_PC)], dst_v.at[b])
        return [
            pltpu.async_copy(xs_hbm.at[src_v.at[b].at[j]],
                             rows_v.at[b].at[pl.ds(j * 128, 128)], sems[b])
            for j in range(ROWS_PC)
        ]

    descs = {0: stage_and_fire(0, 0)}
    sdescs = {}
    for g in range(CHUNKS):
        b = g & 1
        if g - 1 >= 0:
            for d in sdescs.pop(g - 1):
                d.wait()
        if g + 1 < CHUNKS:
            descs[g + 1] = stage_and_fire(g + 1, 1 - b)
        for d in descs.pop(g):
            d.wait()
        sdescs[g] = [
            pltpu.async_copy(rows_v.at[b].at[pl.ds(j * 128, 128)],
                             acc_sh.at[dst_v.at[b].at[j]], sems_s[b], add=True)
            for j in range(ROWS_PC)
        ]
    for d in sdescs.pop(CHUNKS - 1):
        d.wait()

    plsc.subcore_barrier()
    pltpu.sync_copy(acc_sh.at[pl.ds(sid * SLICE, SLICE)],
                    acc_out_hbm.at[cid].at[pl.ds(sid * SLICE, SLICE)])


def _k_dinv_body(degT_ref, dinv_ref):
    deg = jnp.sum(degT_ref[...], axis=1, keepdims=True) + 1.0
    dinv_ref[...] = jnp.broadcast_to(lax.rsqrt(deg), dinv_ref.shape)


def _dinv_call(degT):
    blk = 1024
    return pl.pallas_call(
        _k_dinv_body,
        grid=(NACC // blk,),
        in_specs=[pl.BlockSpec((blk, NC), lambda i: (i, 0))],
        out_specs=pl.BlockSpec((blk, H), lambda i: (i, 0)),
        out_shape=jax.ShapeDtypeStruct((NACC, H), jnp.float32),
    )(degT)


def _k_dense1_body(x_ref, w_ref, dinv_ref, xs_ref):
    xw = jnp.dot(x_ref[...], w_ref[...], preferred_element_type=jnp.float32)
    xs_ref[...] = xw * dinv_ref[...]


def _dense1_call(x, W1, dinv):
    blk = 1000
    return pl.pallas_call(
        _k_dense1_body,
        grid=(N // blk,),
        in_specs=[
            pl.BlockSpec((blk, D_IN), lambda i: (i, 0)),
            pl.BlockSpec((D_IN, H), lambda i: (0, 0)),
            pl.BlockSpec((blk, H), lambda i: (i, 0)),
        ],
        out_specs=pl.BlockSpec((blk, H), lambda i: (i, 0)),
        out_shape=jax.ShapeDtypeStruct((N, H), jnp.float32),
    )(x, W1, dinv)


def _k_dense2_body(acc_ref, xs1_ref, dinv_ref, b1_ref, w2_ref, xs2_ref):
    a = acc_ref[...]
    dinv = dinv_ref[...]
    h = dinv * (a[0] + a[1] + xs1_ref[...]) + b1_ref[...]
    h = jnp.maximum(h, 0.0)
    xs2_ref[...] = jnp.dot(h, w2_ref[...],
                           preferred_element_type=jnp.float32) * dinv


def _dense2_call(acc, xs1, dinv, b1, W2):
    blk = 1000
    return pl.pallas_call(
        _k_dense2_body,
        grid=(N // blk,),
        in_specs=[
            pl.BlockSpec((NC, blk, H), lambda i: (0, i, 0)),
            pl.BlockSpec((blk, H), lambda i: (i, 0)),
            pl.BlockSpec((blk, H), lambda i: (i, 0)),
            pl.BlockSpec((1, H), lambda i: (0, 0)),
            pl.BlockSpec((H, H), lambda i: (0, 0)),
        ],
        out_specs=pl.BlockSpec((blk, H), lambda i: (i, 0)),
        out_shape=jax.ShapeDtypeStruct((N, H), jnp.float32),
    )(acc, xs1, dinv, b1, W2)


def _k_dense3_body(acc_ref, xs2_ref, dinv_ref, b2_ref, out_ref):
    a = acc_ref[...]
    out_ref[...] = dinv_ref[...] * (a[0] + a[1] + xs2_ref[...]) + b2_ref[...]


def _dense3_call(acc, xs2, dinv, b2):
    blk = 1000
    return pl.pallas_call(
        _k_dense3_body,
        grid=(N // blk,),
        in_specs=[
            pl.BlockSpec((NC, blk, H), lambda i: (0, i, 0)),
            pl.BlockSpec((blk, H), lambda i: (i, 0)),
            pl.BlockSpec((blk, H), lambda i: (i, 0)),
            pl.BlockSpec((1, H), lambda i: (0, 0)),
        ],
        out_specs=pl.BlockSpec((blk, H), lambda i: (i, 0)),
        out_shape=jax.ShapeDtypeStruct((N, H), jnp.float32),
    )(acc, xs2, dinv, b2)


def kernel(x, coo, W1, b1, W2, b2):
    src = coo[:, 0]
    dst = coo[:, 1]
    npad = EPAD - E
    src_p = jnp.concatenate([src, jnp.zeros((npad,), jnp.int32)])
    dst_p = jnp.concatenate([dst, jnp.full((npad,), N, jnp.int32)])
    src2d = src_p.reshape(EPAD // 128, 128)
    dst2d = dst_p.reshape(EPAD // 128, 128)

    deg_parts = _k_deg(dst2d)
    dinv = _dinv_call(deg_parts.T)
    dinv_n = dinv[:N]

    xs1 = _dense1_call(x, W1, dinv_n)
    acc1 = _k_scat(xs1, src2d, dst2d)
    xs2 = _dense2_call(acc1, xs1, dinv_n, b1.reshape(1, H), W2)
    acc2 = _k_scat(xs2, src2d, dst2d)
    out = _dense3_call(acc2, xs2, dinv_n, b2.reshape(1, H))
    return out
